# baseline (speedup 1.0000x reference)
import jax
import jax.numpy as jnp
from jax import lax
from jax.experimental import pallas as pl
from jax.experimental.pallas import tpu as pltpu
from jax.experimental.pallas import tpu_sc as plsc

MAX_POS = 77
HIDDEN = 768
LANES = 16
NCORES = 2
NSUB = 16
NW = NCORES * NSUB

CB = 32
COLS = HIDDEN // LANES


def _emb_body(ids_hbm, tok_hbm, pos_hbm, out_hbm, idx_v, pos_v, buf, gsem):
    w = lax.axis_index("s") * NCORES + lax.axis_index("c")
    nch = ids_hbm.shape[1]

    pltpu.sync_copy(ids_hbm.at[w], idx_v)
    pltpu.sync_copy(pos_hbm, pos_v)

    def chunk(c, _):
        row0 = (w * nch + c) * CB
        pltpu.async_copy(tok_hbm.at[idx_v.at[c]], buf, gsem).wait()
        s0 = lax.rem(row0, MAX_POS)

        def row(r, s):
            for j in range(COLS):
                sl = pl.ds(j * LANES, LANES)
                buf[r, sl] += pos_v[s, sl]
            s = s + 1
            return lax.select(s == MAX_POS, 0, s)

        lax.fori_loop(0, CB, row, s0)
        pltpu.sync_copy(buf, out_hbm.at[pl.ds(row0, CB)])
        return 0

    lax.fori_loop(0, nch, chunk, 0)


def kernel(input_ids, token_embedding, position_embedding):
    b, s = input_ids.shape
    rows = b * s
    assert rows % (NW * CB) == 0
    nch = rows // (NW * CB)
    ids2 = input_ids.astype(jnp.int32).reshape(NW, nch, CB)
    run = pl.kernel(
        _emb_body,
        out_type=jax.ShapeDtypeStruct((rows, HIDDEN), jnp.float32),
        mesh=plsc.VectorSubcoreMesh(core_axis_name="c", subcore_axis_name="s"),
        scratch_types=[
            pltpu.VMEM((nch, CB), jnp.int32),
            pltpu.VMEM((MAX_POS, HIDDEN), jnp.float32),
            pltpu.VMEM((CB, HIDDEN), jnp.float32),
            pltpu.SemaphoreType.DMA,
        ],
    )
    out = run(ids2, token_embedding, position_embedding)
    return out.reshape(b, s, HIDDEN)


# speedup vs baseline: 1.3427x; 1.3427x over previous
import jax
import jax.numpy as jnp
from jax import lax
from jax.experimental import pallas as pl
from jax.experimental.pallas import tpu as pltpu
from jax.experimental.pallas import tpu_sc as plsc

MAX_POS = 77
HIDDEN = 768
LANES = 16
NCORES = 2
NSUB = 16
NW = NCORES * NSUB

CB = 32
COLS = HIDDEN // LANES


def _emb_body(ids_hbm, tok_hbm, pos_hbm, out_hbm,
              idx_v, pos_v, buf_a, buf_b, gsem_a, gsem_b, osem_a, osem_b):
    w = lax.axis_index("s") * NCORES + lax.axis_index("c")
    nch = ids_hbm.shape[1]

    pltpu.sync_copy(ids_hbm.at[w], idx_v)
    pltpu.sync_copy(pos_hbm, pos_v)

    def gather_desc(c, buf, sem):
        return pltpu.make_async_copy(tok_hbm.at[idx_v.at[c]], buf, sem)

    def out_desc(c, buf, sem):
        row0 = (w * nch + c) * CB
        return pltpu.make_async_copy(buf, out_hbm.at[pl.ds(row0, CB)], sem)

    def add_pos(c, buf):
        s0 = lax.rem((w * nch + c) * CB, MAX_POS)

        def row(r, s):
            for j in range(COLS):
                sl = pl.ds(j * LANES, LANES)
                plsc.addupdate(buf.at[r, sl], pos_v[s, sl])
            s = s + 1
            return lax.select(s == MAX_POS, 0, s)

        lax.fori_loop(0, CB, row, s0)

    half = nch // 2

    gather_desc(0, buf_a, gsem_a).start()

    def pair(i, _):
        ca = 2 * i
        @pl.when(i > 0)
        def _():
            out_desc(ca - 1, buf_b, osem_b).wait()

        gather_desc(ca + 1, buf_b, gsem_b).start()
        gather_desc(ca, buf_a, gsem_a).wait()
        add_pos(ca, buf_a)
        out_desc(ca, buf_a, osem_a).start()

        gather_desc(ca + 1, buf_b, gsem_b).wait()
        add_pos(ca + 1, buf_b)
        out_desc(ca, buf_a, osem_a).wait()
        gather_desc(ca + 2, buf_a, gsem_a).start()
        out_desc(ca + 1, buf_b, osem_b).start()
        return 0

    lax.fori_loop(0, half, pair, 0)

    last = nch - 1
    gather_desc(last, buf_a, gsem_a).wait()
    add_pos(last, buf_a)
    out_desc(last, buf_a, osem_a).start()
    out_desc(last - 1, buf_b, osem_b).wait()
    out_desc(last, buf_a, osem_a).wait()


def kernel(input_ids, token_embedding, position_embedding):
    b, s = input_ids.shape
    rows = b * s
    assert rows % (NW * CB) == 0
    nch = rows // (NW * CB)
    assert nch % 2 == 1
    ids2 = input_ids.astype(jnp.int32).reshape(NW, nch, CB)
    run = pl.kernel(
        _emb_body,
        out_type=jax.ShapeDtypeStruct((rows, HIDDEN), jnp.float32),
        mesh=plsc.VectorSubcoreMesh(core_axis_name="c", subcore_axis_name="s"),
        scratch_types=[
            pltpu.VMEM((nch, CB), jnp.int32),
            pltpu.VMEM((MAX_POS, HIDDEN), jnp.float32),
            pltpu.VMEM((CB, HIDDEN), jnp.float32),
            pltpu.VMEM((CB, HIDDEN), jnp.float32),
            pltpu.SemaphoreType.DMA,
            pltpu.SemaphoreType.DMA,
            pltpu.SemaphoreType.DMA,
            pltpu.SemaphoreType.DMA,
        ],
    )
    out = run(ids2, token_embedding, position_embedding)
    return out.reshape(b, s, HIDDEN)


# speedup vs baseline: 5.6854x; 4.2342x over previous
import jax
import jax.numpy as jnp
from jax import lax
from jax.experimental import pallas as pl
from jax.experimental.pallas import tpu as pltpu
from jax.experimental.pallas import tpu_sc as plsc

MAX_POS = 77
HIDDEN = 768
LANES = 16
NCORES = 2
NSUB = 16
NW = NCORES * NSUB

CB = 32
COLS = HIDDEN // LANES


def _emb_body(ids_hbm, tok_hbm, pos_hbm, out_hbm,
              idx_v, pos_v, buf_a, buf_b, gsem_a, gsem_b, osem_a, osem_b):
    w = lax.axis_index("s") * NCORES + lax.axis_index("c")
    nch = ids_hbm.shape[1]
    batch = out_hbm.shape[0] // MAX_POS

    pltpu.sync_copy(ids_hbm.at[w], idx_v)
    pltpu.sync_copy(pos_hbm, pos_v)

    def gather_desc(c, buf, sem):
        return pltpu.make_async_copy(tok_hbm.at[idx_v.at[c]], buf, sem)

    def out_desc(c, buf, sem):
        row0 = (w * nch + c) * CB
        return pltpu.make_async_copy(buf, out_hbm.at[pl.ds(row0, CB)], sem)

    def add_pos(c, buf):
        s = lax.div((w * nch + c) * CB, batch)
        pos_regs = [pos_v[s, pl.ds(j * LANES, LANES)] for j in range(COLS)]

        def row(r, _):
            for j in range(COLS):
                plsc.addupdate(buf.at[r, pl.ds(j * LANES, LANES)], pos_regs[j])
            return 0

        lax.fori_loop(0, CB, row, 0)

    half = nch // 2

    gather_desc(0, buf_a, gsem_a).start()

    def pair(i, _):
        ca = 2 * i
        @pl.when(i > 0)
        def _():
            out_desc(ca - 1, buf_b, osem_b).wait()

        gather_desc(ca + 1, buf_b, gsem_b).start()
        gather_desc(ca, buf_a, gsem_a).wait()
        add_pos(ca, buf_a)
        out_desc(ca, buf_a, osem_a).start()

        gather_desc(ca + 1, buf_b, gsem_b).wait()
        add_pos(ca + 1, buf_b)
        out_desc(ca, buf_a, osem_a).wait()
        gather_desc(ca + 2, buf_a, gsem_a).start()
        out_desc(ca + 1, buf_b, osem_b).start()
        return 0

    lax.fori_loop(0, half, pair, 0)

    last = nch - 1
    gather_desc(last, buf_a, gsem_a).wait()
    add_pos(last, buf_a)
    out_desc(last, buf_a, osem_a).start()
    out_desc(last - 1, buf_b, osem_b).wait()
    out_desc(last, buf_a, osem_a).wait()


def kernel(input_ids, token_embedding, position_embedding):
    b, s = input_ids.shape
    rows = b * s
    assert rows % (NW * CB) == 0 and b % CB == 0
    nch = rows // (NW * CB)
    assert nch % 2 == 1
    ids2 = input_ids.T.astype(jnp.int32).reshape(NW, nch, CB)
    run = pl.kernel(
        _emb_body,
        out_type=jax.ShapeDtypeStruct((rows, HIDDEN), jnp.float32),
        mesh=plsc.VectorSubcoreMesh(core_axis_name="c", subcore_axis_name="s"),
        scratch_types=[
            pltpu.VMEM((nch, CB), jnp.int32),
            pltpu.VMEM((MAX_POS, HIDDEN), jnp.float32),
            pltpu.VMEM((CB, HIDDEN), jnp.float32),
            pltpu.VMEM((CB, HIDDEN), jnp.float32),
            pltpu.SemaphoreType.DMA,
            pltpu.SemaphoreType.DMA,
            pltpu.SemaphoreType.DMA,
            pltpu.SemaphoreType.DMA,
        ],
    )
    out = run(ids2, token_embedding, position_embedding)
    return jnp.swapaxes(out.reshape(s, b, HIDDEN), 0, 1)


# speedup vs baseline: 6.0015x; 1.0556x over previous
import jax
import jax.numpy as jnp
from jax import lax
from jax.experimental import pallas as pl
from jax.experimental.pallas import tpu as pltpu
from jax.experimental.pallas import tpu_sc as plsc

MAX_POS = 77
HIDDEN = 768
LANES = 16
NCORES = 2
NSUB = 16
NW = NCORES * NSUB

CB = 32
COLS = HIDDEN // LANES
NBUF = 4


def _emb_body(ids_hbm, tok_hbm, pos_hbm, out_hbm,
              idx_v, pos_v, b0, b1, b2, b3,
              g0, g1, g2, g3, o0, o1, o2, o3, psem):
    bufs = (b0, b1, b2, b3)
    gsems = (g0, g1, g2, g3)
    osems = (o0, o1, o2, o3)
    w = lax.axis_index("s") * NCORES + lax.axis_index("c")
    nch = ids_hbm.shape[1]
    batch = out_hbm.shape[0] // MAX_POS

    pltpu.sync_copy(ids_hbm.at[w], idx_v)
    pbase = lax.min(lax.div(w * nch * CB, batch), MAX_POS - 16)
    pidx = pbase + lax.iota(jnp.int32, 16)
    pltpu.async_copy(pos_hbm.at[pidx], pos_v, psem).wait()

    def gather_desc(c, buf, sem):
        return pltpu.make_async_copy(tok_hbm.at[idx_v.at[c]], buf, sem)

    def out_desc(c, buf, sem):
        row0 = (w * nch + c) * CB
        return pltpu.make_async_copy(buf, out_hbm.at[pl.ds(row0, CB)], sem)

    def add_pos(c, buf):
        srow = lax.div((w * nch + c) * CB, batch) - pbase
        pos_regs = [pos_v[srow, pl.ds(j * LANES, LANES)] for j in range(COLS)]

        def rows(r, _):
            for u in range(2):
                for j in range(COLS):
                    plsc.addupdate(
                        buf.at[2 * r + u, pl.ds(j * LANES, LANES)], pos_regs[j])
            return 0

        lax.fori_loop(0, CB // 2, rows, 0)

    def step(c, k):
        @pl.when(c >= 2)
        def _():
            out_desc(c - 2, bufs[(k + 2) % NBUF], osems[(k + 2) % NBUF]).wait()

        @pl.when(c <= nch - 3)
        def _():
            gather_desc(c + 2, bufs[(k + 2) % NBUF], gsems[(k + 2) % NBUF]).start()

        gather_desc(c, bufs[k], gsems[k]).wait()
        add_pos(c, bufs[k])
        out_desc(c, bufs[k], osems[k]).start()

    gather_desc(0, bufs[0], gsems[0]).start()
    gather_desc(1, bufs[1], gsems[1]).start()

    def quad(t, _):
        for k in range(NBUF):
            step(NBUF * t + k, k)
        return 0

    lax.fori_loop(0, nch // NBUF, quad, 0)
    step(nch - 1, (nch - 1) % NBUF)

    out_desc(nch - 2, bufs[(nch - 2) % NBUF], osems[(nch - 2) % NBUF]).wait()
    out_desc(nch - 1, bufs[(nch - 1) % NBUF], osems[(nch - 1) % NBUF]).wait()


def kernel(input_ids, token_embedding, position_embedding):
    b, s = input_ids.shape
    rows = b * s
    assert rows % (NW * CB) == 0 and b % CB == 0
    nch = rows // (NW * CB)
    assert nch % NBUF == 1
    ids2 = input_ids.T.astype(jnp.int32).reshape(NW, nch, CB)
    run = pl.kernel(
        _emb_body,
        out_type=jax.ShapeDtypeStruct((rows, HIDDEN), jnp.float32),
        mesh=plsc.VectorSubcoreMesh(core_axis_name="c", subcore_axis_name="s"),
        scratch_types=[
            pltpu.VMEM((nch, CB), jnp.int32),
            pltpu.VMEM((16, HIDDEN), jnp.float32),
            pltpu.VMEM((CB, HIDDEN), jnp.float32),
            pltpu.VMEM((CB, HIDDEN), jnp.float32),
            pltpu.VMEM((CB, HIDDEN), jnp.float32),
            pltpu.VMEM((CB, HIDDEN), jnp.float32),
        ] + [pltpu.SemaphoreType.DMA] * 9,
    )
    out = run(ids2, token_embedding, position_embedding)
    return jnp.swapaxes(out.reshape(s, b, HIDDEN), 0, 1)
